# baseline (speedup 1.0000x reference)
import jax
import jax.numpy as jnp
from jax.experimental import pallas as pl
from jax.experimental.pallas import tpu as pltpu

B = 16384
D = 3072
NUM_CLASSES = 10
NCHUNK = 8
ROWS = B // NCHUNK


def _body(x_hbm, y_ref, xout_hbm, oh_ref, sems):
    for i in range(NCHUNK):
        pltpu.make_async_copy(
            x_hbm.at[pl.ds(i * ROWS, ROWS), :],
            xout_hbm.at[pl.ds(i * ROWS, ROWS), :],
            sems.at[i],
        ).start()
    yv = y_ref[...]
    iota = jax.lax.broadcasted_iota(jnp.int32, (B, NUM_CLASSES), 1)
    oh_ref[...] = (yv == iota).astype(jnp.float32)
    for i in range(NCHUNK):
        pltpu.make_async_copy(
            x_hbm.at[pl.ds(i * ROWS, ROWS), :],
            xout_hbm.at[pl.ds(i * ROWS, ROWS), :],
            sems.at[i],
        ).wait()


def kernel(x, y):
    y2 = y.reshape(B, 1)
    x_out, one_hot = pl.pallas_call(
        _body,
        in_specs=[
            pl.BlockSpec(memory_space=pltpu.HBM),
            pl.BlockSpec(memory_space=pltpu.VMEM),
        ],
        out_specs=[
            pl.BlockSpec(memory_space=pltpu.HBM),
            pl.BlockSpec(memory_space=pltpu.VMEM),
        ],
        out_shape=[
            jax.ShapeDtypeStruct((B, D), jnp.float32),
            jax.ShapeDtypeStruct((B, NUM_CLASSES), jnp.float32),
        ],
        scratch_shapes=[pltpu.SemaphoreType.DMA((NCHUNK,))],
    )(x, y2)
    return (x_out, one_hot)


# speedup vs baseline: 43.3651x; 43.3651x over previous
import jax
import jax.numpy as jnp
from jax.experimental import pallas as pl
from jax.experimental.pallas import tpu as pltpu

B = 16384
D = 3072
NUM_CLASSES = 10
RB = 1024
NBLK = B // RB


def _body(x_ref, y_ref, xout_ref, oh_ref):
    xout_ref[...] = x_ref[...]
    yv = y_ref[...]
    iota = jax.lax.broadcasted_iota(jnp.int32, (RB, NUM_CLASSES), 1)
    oh_ref[...] = (yv == iota).astype(jnp.float32)


def kernel(x, y):
    y2 = y.reshape(B, 1)
    x_out, one_hot = pl.pallas_call(
        _body,
        grid=(NBLK,),
        in_specs=[
            pl.BlockSpec((RB, D), lambda i: (i, 0)),
            pl.BlockSpec((RB, 1), lambda i: (i, 0)),
        ],
        out_specs=[
            pl.BlockSpec((RB, D), lambda i: (i, 0)),
            pl.BlockSpec((RB, NUM_CLASSES), lambda i: (i, 0)),
        ],
        out_shape=[
            jax.ShapeDtypeStruct((B, D), jnp.float32),
            jax.ShapeDtypeStruct((B, NUM_CLASSES), jnp.float32),
        ],
        compiler_params=pltpu.CompilerParams(
            dimension_semantics=("arbitrary",),
        ),
    )(x, y2)
    return (x_out, one_hot)
